# baseline (speedup 1.0000x reference)
import functools

import jax
import jax.numpy as jnp
from jax.experimental import pallas as pl
from jax.experimental.pallas import tpu as pltpu

HEADS = 12
DQK = 64
DV = 64
SPAN = 32
LQ = 2048
DIM = 768

_dot = functools.partial(jax.lax.dot_general,
                         preferred_element_type=jnp.float32)


def _proj_kernel(q_ref, k_ref, v_ref, wq_ref, wk_ref, wv_ref,
                 qt_ref, kt_ref, vt_ref):
    dn = (((1,), (1,)), ((), ()))
    qt_ref[...] = _dot(wq_ref[...], q_ref[...], dimension_numbers=dn)
    kt_ref[...] = _dot(wk_ref[...], k_ref[...], dimension_numbers=dn)
    vt_ref[...] = _dot(wv_ref[...], v_ref[...], dimension_numbers=dn)


_BLK = 128
_NBLK = LQ // _BLK
_HPS = 2
_STEPS = HEADS // _HPS


def _band_one_head(qt, kt, vt):
    zpad = jnp.zeros((DQK, SPAN - 1), jnp.float32)
    kt0 = jnp.concatenate([zpad, jax.lax.slice_in_dim(kt, 0, _BLK, axis=1)],
                          axis=1)
    vt0 = jnp.concatenate([zpad, jax.lax.slice_in_dim(vt, 0, _BLK, axis=1)],
                          axis=1)

    scale = 1.0 / (DQK ** 0.5)
    s_blocks = []
    for t in range(_NBLK):
        qtb = jax.lax.slice_in_dim(qt, t * _BLK, (t + 1) * _BLK, axis=1)
        rows = []
        for i in range(SPAN):
            if t == 0:
                ks = jax.lax.slice_in_dim(kt0, i, i + _BLK, axis=1)
            else:
                ks = jax.lax.slice_in_dim(kt, t * _BLK + i - (SPAN - 1),
                                          t * _BLK + i - (SPAN - 1) + _BLK,
                                          axis=1)
            rows.append(jnp.sum(qtb * ks, axis=0, keepdims=True))
        s_blocks.append(jnp.concatenate(rows, axis=0))
    s = jnp.concatenate(s_blocks, axis=1) * scale

    iidx = jax.lax.broadcasted_iota(jnp.int32, (SPAN, LQ), 0)
    jidx = jax.lax.broadcasted_iota(jnp.int32, (SPAN, LQ), 1)
    s = jnp.where(iidx + jidx >= SPAN - 1, s, -jnp.inf)

    m = jnp.max(s, axis=1, keepdims=True)
    e = jnp.exp(s - m)
    w = e / jnp.sum(e, axis=1, keepdims=True)

    out_blocks = []
    for t in range(_NBLK):
        accb = jnp.zeros((DV, _BLK), jnp.float32)
        wb = jax.lax.slice_in_dim(w, t * _BLK, (t + 1) * _BLK, axis=1)
        for i in range(SPAN):
            if t == 0:
                vs = jax.lax.slice_in_dim(vt0, i, i + _BLK, axis=1)
            else:
                vs = jax.lax.slice_in_dim(vt, t * _BLK + i - (SPAN - 1),
                                          t * _BLK + i - (SPAN - 1) + _BLK,
                                          axis=1)
            accb = accb + wb[i:i + 1, :] * vs
        out_blocks.append(accb)
    return jnp.concatenate(out_blocks, axis=1)


def _band_kernel(qt_ref, kt_ref, vt_ref, wo_ref, out_ref, qkvt_ref):
    p = pl.program_id(0)
    for hh in range(_HPS):
        sl = pl.ds(hh * DQK, DQK)
        acc = _band_one_head(qt_ref[sl, :], kt_ref[sl, :], vt_ref[sl, :])
        qkvt_ref[pl.ds(p * (_HPS * DV) + hh * DV, DV), :] = acc

    @pl.when(p == _STEPS - 1)
    def _():
        out_ref[...] = _dot(qkvt_ref[...], wo_ref[...],
                            dimension_numbers=(((0,), (1,)), ((), ())))


def kernel(q, k, v, Wq, Wk, Wv, Wo):
    b, lq, dim_q = q.shape
    q2 = q.reshape(lq, dim_q)
    k2 = k.reshape(lq, dim_q)
    v2 = v.reshape(lq, dim_q)

    qt, kt, vt = pl.pallas_call(
        _proj_kernel,
        grid=(1,),
        in_specs=[pl.BlockSpec((LQ, DIM), lambda i: (0, 0))] * 3
        + [pl.BlockSpec((DIM, DIM), lambda i: (0, 0))] * 3,
        out_specs=[pl.BlockSpec((DIM, LQ), lambda i: (0, 0))] * 3,
        out_shape=[jax.ShapeDtypeStruct((DIM, LQ), jnp.float32)] * 3,
    )(q2, k2, v2, Wq, Wk, Wv)

    out = pl.pallas_call(
        _band_kernel,
        grid=(_STEPS,),
        in_specs=[
            pl.BlockSpec((_HPS * DQK, LQ), lambda h: (h, 0)),
            pl.BlockSpec((_HPS * DQK, LQ), lambda h: (h, 0)),
            pl.BlockSpec((_HPS * DV, LQ), lambda h: (h, 0)),
            pl.BlockSpec((DIM, HEADS * DV), lambda h: (0, 0)),
        ],
        out_specs=pl.BlockSpec((LQ, DIM), lambda h: (0, 0)),
        out_shape=jax.ShapeDtypeStruct((LQ, DIM), jnp.float32),
        scratch_shapes=[pltpu.VMEM((HEADS * DV, LQ), jnp.float32)],
    )(qt, kt, vt, Wo)

    return out.reshape(b, lq, DIM)


# speedup vs baseline: 1.0020x; 1.0020x over previous
import functools

import jax
import jax.numpy as jnp
from jax.experimental import pallas as pl
from jax.experimental.pallas import tpu as pltpu

HEADS = 12
DQK = 64
DV = 64
SPAN = 32
LQ = 2048
DIM = 768

_dot = functools.partial(jax.lax.dot_general,
                         preferred_element_type=jnp.float32)


def _proj_kernel(q_ref, k_ref, v_ref, wq_ref, wk_ref, wv_ref,
                 qt_ref, kt_ref, vt_ref):
    dn = (((1,), (1,)), ((), ()))
    qt_ref[...] = _dot(wq_ref[...], q_ref[...], dimension_numbers=dn)
    kt_ref[...] = _dot(wk_ref[...], k_ref[...], dimension_numbers=dn)
    vt_ref[...] = _dot(wv_ref[...], v_ref[...], dimension_numbers=dn)


_BLK = 128
_NBLK = LQ // _BLK
_HPS = 1
_STEPS = HEADS // _HPS


def _band_one_head(qt, kt, vt):
    zpad = jnp.zeros((DQK, SPAN - 1), jnp.float32)
    kt0 = jnp.concatenate([zpad, jax.lax.slice_in_dim(kt, 0, _BLK, axis=1)],
                          axis=1)
    vt0 = jnp.concatenate([zpad, jax.lax.slice_in_dim(vt, 0, _BLK, axis=1)],
                          axis=1)

    scale = 1.0 / (DQK ** 0.5)
    s_blocks = []
    for t in range(_NBLK):
        qtb = jax.lax.slice_in_dim(qt, t * _BLK, (t + 1) * _BLK, axis=1)
        rows = []
        for i in range(SPAN):
            if t == 0:
                ks = jax.lax.slice_in_dim(kt0, i, i + _BLK, axis=1)
            else:
                ks = jax.lax.slice_in_dim(kt, t * _BLK + i - (SPAN - 1),
                                          t * _BLK + i - (SPAN - 1) + _BLK,
                                          axis=1)
            rows.append(jnp.sum(qtb * ks, axis=0, keepdims=True))
        s_blocks.append(jnp.concatenate(rows, axis=0))
    s = jnp.concatenate(s_blocks, axis=1) * scale

    iidx = jax.lax.broadcasted_iota(jnp.int32, (SPAN, LQ), 0)
    jidx = jax.lax.broadcasted_iota(jnp.int32, (SPAN, LQ), 1)
    s = jnp.where(iidx + jidx >= SPAN - 1, s, -jnp.inf)

    m = jnp.max(s, axis=1, keepdims=True)
    e = jnp.exp(s - m)
    w = e / jnp.sum(e, axis=1, keepdims=True)

    out_blocks = []
    for t in range(_NBLK):
        accb = jnp.zeros((DV, _BLK), jnp.float32)
        wb = jax.lax.slice_in_dim(w, t * _BLK, (t + 1) * _BLK, axis=1)
        for i in range(SPAN):
            if t == 0:
                vs = jax.lax.slice_in_dim(vt0, i, i + _BLK, axis=1)
            else:
                vs = jax.lax.slice_in_dim(vt, t * _BLK + i - (SPAN - 1),
                                          t * _BLK + i - (SPAN - 1) + _BLK,
                                          axis=1)
            accb = accb + wb[i:i + 1, :] * vs
        out_blocks.append(accb)
    return jnp.concatenate(out_blocks, axis=1)


def _band_kernel(qt_ref, kt_ref, vt_ref, wo_ref, out_ref, qkvt_ref):
    p = pl.program_id(0)
    for hh in range(_HPS):
        sl = pl.ds(hh * DQK, DQK)
        acc = _band_one_head(qt_ref[sl, :], kt_ref[sl, :], vt_ref[sl, :])
        qkvt_ref[pl.ds(p * (_HPS * DV) + hh * DV, DV), :] = acc

    @pl.when(p == _STEPS - 1)
    def _():
        out_ref[...] = _dot(qkvt_ref[...], wo_ref[...],
                            dimension_numbers=(((0,), (1,)), ((), ())))


def kernel(q, k, v, Wq, Wk, Wv, Wo):
    b, lq, dim_q = q.shape
    q2 = q.reshape(lq, dim_q)
    k2 = k.reshape(lq, dim_q)
    v2 = v.reshape(lq, dim_q)

    qt, kt, vt = pl.pallas_call(
        _proj_kernel,
        grid=(1,),
        in_specs=[pl.BlockSpec((LQ, DIM), lambda i: (0, 0))] * 3
        + [pl.BlockSpec((DIM, DIM), lambda i: (0, 0))] * 3,
        out_specs=[pl.BlockSpec((DIM, LQ), lambda i: (0, 0))] * 3,
        out_shape=[jax.ShapeDtypeStruct((DIM, LQ), jnp.float32)] * 3,
    )(q2, k2, v2, Wq, Wk, Wv)

    out = pl.pallas_call(
        _band_kernel,
        grid=(_STEPS,),
        in_specs=[
            pl.BlockSpec((_HPS * DQK, LQ), lambda h: (h, 0)),
            pl.BlockSpec((_HPS * DQK, LQ), lambda h: (h, 0)),
            pl.BlockSpec((_HPS * DV, LQ), lambda h: (h, 0)),
            pl.BlockSpec((DIM, HEADS * DV), lambda h: (0, 0)),
        ],
        out_specs=pl.BlockSpec((LQ, DIM), lambda h: (0, 0)),
        out_shape=jax.ShapeDtypeStruct((LQ, DIM), jnp.float32),
        scratch_shapes=[pltpu.VMEM((HEADS * DV, LQ), jnp.float32)],
    )(qt, kt, vt, Wo)

    return out.reshape(b, lq, DIM)


# speedup vs baseline: 1.0482x; 1.0462x over previous
import functools

import jax
import jax.numpy as jnp
from jax.experimental import pallas as pl
from jax.experimental.pallas import tpu as pltpu

HEADS = 12
DQK = 64
DV = 64
SPAN = 32
LQ = 2048
DIM = 768

_dot = functools.partial(jax.lax.dot_general,
                         preferred_element_type=jnp.float32)


def _proj_kernel(q_ref, k_ref, v_ref, wq_ref, wk_ref, wv_ref,
                 qt_ref, kt_ref, vt_ref):
    dn = (((1,), (1,)), ((), ()))
    qt_ref[...] = _dot(wq_ref[...], q_ref[...], dimension_numbers=dn)
    kt_ref[...] = _dot(wk_ref[...], k_ref[...], dimension_numbers=dn)
    vt_ref[...] = _dot(wv_ref[...], v_ref[...], dimension_numbers=dn)


_BLK = 256
_NBLK = LQ // _BLK


def _band_kernel(qt_ref, kt_ref, vt_ref, wo_ref, out_ref, qkvt_ref):
    h = pl.program_id(0)
    qt = qt_ref[...]
    kt = kt_ref[...]
    vt = vt_ref[...]

    zpad = jnp.zeros((DQK, SPAN - 1), jnp.float32)
    ktp = jnp.concatenate([zpad, kt], axis=1)
    vtp = jnp.concatenate([zpad, vt], axis=1)

    scale = 1.0 / (DQK ** 0.5)
    s_blocks = []
    for t in range(_NBLK):
        qtb = jax.lax.slice_in_dim(qt, t * _BLK, (t + 1) * _BLK, axis=1)
        rows = []
        for i in range(SPAN):
            ks = jax.lax.slice_in_dim(ktp, t * _BLK + i, t * _BLK + i + _BLK,
                                      axis=1)
            rows.append(jnp.sum(qtb * ks, axis=0, keepdims=True))
        s_blocks.append(jnp.concatenate(rows, axis=0))
    s = jnp.concatenate(s_blocks, axis=1) * scale

    iidx = jax.lax.broadcasted_iota(jnp.int32, (SPAN, LQ), 0)
    jidx = jax.lax.broadcasted_iota(jnp.int32, (SPAN, LQ), 1)
    s = jnp.where(iidx + jidx >= SPAN - 1, s, -jnp.inf)

    m = jnp.max(s, axis=1, keepdims=True)
    e = jnp.exp(s - m)
    w = e / jnp.sum(e, axis=1, keepdims=True)

    out_blocks = []
    for t in range(_NBLK):
        accb = jnp.zeros((DV, _BLK), jnp.float32)
        wb = jax.lax.slice_in_dim(w, t * _BLK, (t + 1) * _BLK, axis=1)
        for i in range(SPAN):
            vs = jax.lax.slice_in_dim(vtp, t * _BLK + i, t * _BLK + i + _BLK,
                                      axis=1)
            accb = accb + wb[i:i + 1, :] * vs
        out_blocks.append(accb)
    acc = jnp.concatenate(out_blocks, axis=1)

    qkvt_ref[pl.ds(h * DV, DV), :] = acc

    @pl.when(h == HEADS - 1)
    def _():
        out_ref[...] = _dot(qkvt_ref[...], wo_ref[...],
                            dimension_numbers=(((0,), (1,)), ((), ())))


def kernel(q, k, v, Wq, Wk, Wv, Wo):
    b, lq, dim_q = q.shape
    q2 = q.reshape(lq, dim_q)
    k2 = k.reshape(lq, dim_q)
    v2 = v.reshape(lq, dim_q)

    qt, kt, vt = pl.pallas_call(
        _proj_kernel,
        grid=(1,),
        in_specs=[pl.BlockSpec((LQ, DIM), lambda i: (0, 0))] * 3
        + [pl.BlockSpec((DIM, DIM), lambda i: (0, 0))] * 3,
        out_specs=[pl.BlockSpec((DIM, LQ), lambda i: (0, 0))] * 3,
        out_shape=[jax.ShapeDtypeStruct((DIM, LQ), jnp.float32)] * 3,
    )(q2, k2, v2, Wq, Wk, Wv)

    out = pl.pallas_call(
        _band_kernel,
        grid=(HEADS,),
        in_specs=[
            pl.BlockSpec((DQK, LQ), lambda h: (h, 0)),
            pl.BlockSpec((DQK, LQ), lambda h: (h, 0)),
            pl.BlockSpec((DV, LQ), lambda h: (h, 0)),
            pl.BlockSpec((DIM, HEADS * DV), lambda h: (0, 0)),
        ],
        out_specs=pl.BlockSpec((LQ, DIM), lambda h: (0, 0)),
        out_shape=jax.ShapeDtypeStruct((LQ, DIM), jnp.float32),
        scratch_shapes=[pltpu.VMEM((HEADS * DV, LQ), jnp.float32)],
    )(qt, kt, vt, Wo)

    return out.reshape(b, lq, DIM)
